# Initial kernel scaffold; baseline (speedup 1.0000x reference)
import jax
import jax.numpy as jnp
from jax.experimental import pallas as pl


def kernel(weights, starts, ends):
    raise NotImplementedError("write your pallas kernel here")


# speedup vs baseline: 15.4885x; 15.4885x over previous
import jax
import jax.numpy as jnp
from jax.experimental import pallas as pl

_R = 2048


def _body(w_ref, s_ref, e_ref, o_ref):
    w = w_ref[...]
    r, s = w.shape
    tri = (
        jax.lax.broadcasted_iota(jnp.int32, (s, s), 0)
        <= jax.lax.broadcasted_iota(jnp.int32, (s, s), 1)
    ).astype(jnp.float32)
    cum = jnp.dot(
        w, tri, preferred_element_type=jnp.float32,
        precision=jax.lax.Precision.HIGHEST,
    )
    cnt = jnp.sum((cum < 0.5).astype(jnp.int32), axis=1, keepdims=True)
    cnt = jnp.minimum(cnt, s - 1)
    lane = jax.lax.broadcasted_iota(jnp.int32, (r, s), 1)
    sel = lane == cnt
    steps = (s_ref[...] + e_ref[...]) * 0.5
    o_ref[...] = jnp.sum(jnp.where(sel, steps, 0.0), axis=1, keepdims=True)


def kernel(weights, starts, ends):
    B, S = weights.shape[0], weights.shape[1]
    w2 = weights.reshape(B, S)
    s2 = starts.reshape(B, S)
    e2 = ends.reshape(B, S)
    out = pl.pallas_call(
        _body,
        grid=(B // _R,),
        in_specs=[pl.BlockSpec((_R, S), lambda i: (i, 0))] * 3,
        out_specs=pl.BlockSpec((_R, 1), lambda i: (i, 0)),
        out_shape=jax.ShapeDtypeStruct((B, 1), jnp.float32),
    )(w2, s2, e2)
    return out


# speedup vs baseline: 17.8465x; 1.1522x over previous
import functools

import jax
import jax.numpy as jnp
from jax import lax
from jax.experimental import pallas as pl
from jax.experimental.pallas import tpu as pltpu
from jax.experimental.pallas import tpu_sc as plsc

_R = 2048

_NW = 32
_CPR = 8


def _idx_body(w_ref, o_ref):
    w = w_ref[...]
    r, s = w.shape
    tri = (
        lax.broadcasted_iota(jnp.int32, (s, s), 0)
        <= lax.broadcasted_iota(jnp.int32, (s, s), 1)
    ).astype(jnp.float32)
    cum = jnp.dot(
        w, tri, preferred_element_type=jnp.float32,
        precision=lax.Precision.HIGHEST,
    )
    cnt = jnp.sum((cum < 0.5).astype(jnp.int32), axis=1, keepdims=True)
    cnt = jnp.minimum(cnt, s - 1)
    row = lax.broadcasted_iota(jnp.int32, (r, 1), 0) + pl.program_id(0) * r
    o_ref[...] = row * s + cnt


def _median_indices(w2):
    B, S = w2.shape
    return pl.pallas_call(
        _idx_body,
        grid=(B // _R,),
        in_specs=[pl.BlockSpec((_R, S), lambda i: (i, 0))],
        out_specs=pl.BlockSpec((_R, 1), lambda i: (i, 0)),
        out_shape=jax.ShapeDtypeStruct((B, 1), jnp.int32),
    )(w2)


def _make_sc_gather(B):
    bpw = B // _NW
    nch = bpw // 128
    nrounds = nch // _CPR
    mesh = plsc.VectorSubcoreMesh(core_axis_name="c", subcore_axis_name="s")

    @functools.partial(
        pl.kernel,
        out_type=jax.ShapeDtypeStruct((_NW, nch, 128), jnp.float32),
        mesh=mesh,
        scratch_types=[
            pltpu.VMEM((nch, 128), jnp.int32),
            pltpu.VMEM((nch, 128), jnp.float32),
            pltpu.VMEM((nch, 128), jnp.float32),
            pltpu.SemaphoreType.DMA,
            pltpu.SemaphoreType.DMA,
        ],
    )
    def gather_kernel(gidx_hbm, s_hbm, e_hbm, out_hbm,
                      idx_v, sbuf, ebuf, sem_s, sem_e):
        wid = lax.axis_index("s") * 2 + lax.axis_index("c")
        pltpu.sync_copy(gidx_hbm.at[wid], idx_v)

        def gather_round(r, carry):
            handles = []
            for j in range(_CPR):
                c = r * _CPR + j
                handles.append(
                    pltpu.async_copy(s_hbm.at[idx_v.at[c]], sbuf.at[c], sem_s))
                handles.append(
                    pltpu.async_copy(e_hbm.at[idx_v.at[c]], ebuf.at[c], sem_e))
            for h in handles:
                h.wait()
            return carry

        lax.fori_loop(0, nrounds, gather_round, 0)

        def avg_row(i, carry):
            for j in range(8):
                sl = pl.ds(j * 16, 16)
                sbuf[i, sl] = (sbuf[i, sl] + ebuf[i, sl]) * 0.5
            return carry

        lax.fori_loop(0, nch, avg_row, 0)

        pltpu.sync_copy(sbuf, out_hbm.at[wid])

    return gather_kernel


def kernel(weights, starts, ends):
    B, S = weights.shape[0], weights.shape[1]
    w2 = weights.reshape(B, S)
    gidx = _median_indices(w2)
    gidx3 = gidx.reshape(_NW, -1, 128)
    sflat = starts.reshape(B * S)
    eflat = ends.reshape(B * S)
    out = _make_sc_gather(B)(gidx3, sflat, eflat)
    return out.reshape(B, 1)


# speedup vs baseline: 21.7080x; 1.2164x over previous
import functools

import jax
import jax.numpy as jnp
from jax import lax
from jax.experimental import pallas as pl
from jax.experimental.pallas import tpu as pltpu
from jax.experimental.pallas import tpu_sc as plsc

_R = 4096

_NW = 32
_CPR = 8


def _idx_body(w_ref, o_ref):
    w = w_ref[...]
    r, s = w.shape
    tri = (
        lax.broadcasted_iota(jnp.int32, (s, s), 0)
        <= lax.broadcasted_iota(jnp.int32, (s, s), 1)
    ).astype(jnp.float32)
    cum = jnp.dot(
        w, tri, preferred_element_type=jnp.float32,
        precision=lax.Precision.HIGHEST,
    )
    cntf = jnp.sum(jnp.where(cum < 0.5, 1.0, 0.0), axis=1, keepdims=True)
    cnt = jnp.minimum(cntf.astype(jnp.int32), s - 1)
    row = lax.broadcasted_iota(jnp.int32, (r, 1), 0) + pl.program_id(0) * r
    o_ref[...] = row * s + cnt


def _median_indices(w2):
    B, S = w2.shape
    return pl.pallas_call(
        _idx_body,
        grid=(B // _R,),
        in_specs=[pl.BlockSpec((_R, S), lambda i: (i, 0))],
        out_specs=pl.BlockSpec((_R, 1), lambda i: (i, 0)),
        out_shape=jax.ShapeDtypeStruct((B, 1), jnp.int32),
    )(w2)


def _make_sc_gather(B):
    bpw = B // _NW
    nch = bpw // 128
    nrounds = nch // _CPR
    mesh = plsc.VectorSubcoreMesh(core_axis_name="c", subcore_axis_name="s")

    @functools.partial(
        pl.kernel,
        out_type=jax.ShapeDtypeStruct((_NW, nch, 128), jnp.float32),
        mesh=mesh,
        scratch_types=[
            pltpu.VMEM((nch, 128), jnp.int32),
            pltpu.VMEM((nch, 128), jnp.float32),
            pltpu.VMEM((nch, 128), jnp.float32),
            pltpu.SemaphoreType.DMA,
            pltpu.SemaphoreType.DMA,
        ],
    )
    def gather_kernel(gidx_hbm, s_hbm, e_hbm, out_hbm,
                      idx_v, sbuf, ebuf, sem_s, sem_e):
        wid = lax.axis_index("s") * 2 + lax.axis_index("c")
        pltpu.sync_copy(gidx_hbm.at[wid], idx_v)

        def gather_round(r, carry):
            handles = []
            for j in range(_CPR):
                c = r * _CPR + j
                handles.append(
                    pltpu.async_copy(s_hbm.at[idx_v.at[c]], sbuf.at[c], sem_s))
                handles.append(
                    pltpu.async_copy(e_hbm.at[idx_v.at[c]], ebuf.at[c], sem_e))
            for h in handles:
                h.wait()
            return carry

        lax.fori_loop(0, nrounds, gather_round, 0)

        def avg_row(i, carry):
            for j in range(8):
                sl = pl.ds(j * 16, 16)
                sbuf[i, sl] = (sbuf[i, sl] + ebuf[i, sl]) * 0.5
            return carry

        lax.fori_loop(0, nch, avg_row, 0)

        pltpu.sync_copy(sbuf, out_hbm.at[wid])

    return gather_kernel


def kernel(weights, starts, ends):
    B, S = weights.shape[0], weights.shape[1]
    w2 = weights.reshape(B, S)
    gidx = _median_indices(w2)
    gidx3 = gidx.reshape(_NW, -1, 128)
    sflat = starts.reshape(B * S)
    eflat = ends.reshape(B * S)
    out = _make_sc_gather(B)(gidx3, sflat, eflat)
    return out.reshape(B, 1)


# speedup vs baseline: 22.6550x; 1.0436x over previous
import functools

import jax
import jax.numpy as jnp
from jax import lax
from jax.experimental import pallas as pl
from jax.experimental.pallas import tpu as pltpu
from jax.experimental.pallas import tpu_sc as plsc

_R = 2048
_Q = 4

_NW = 32
_CPR = 8


def _idx_body(*refs):
    w_refs, o_ref = refs[:_Q], refs[_Q]
    s = w_refs[0].shape[1]
    r = w_refs[0].shape[0]
    tri = (
        lax.broadcasted_iota(jnp.int32, (s, s), 0)
        <= lax.broadcasted_iota(jnp.int32, (s, s), 1)
    ).astype(jnp.float32)
    for q in range(_Q):
        w = w_refs[q][...]
        cum = jnp.dot(
            w, tri, preferred_element_type=jnp.float32,
            precision=lax.Precision.HIGHEST,
        )
        cntf = jnp.sum(jnp.where(cum < 0.5, 1.0, 0.0), axis=1, keepdims=True)
        cnt = jnp.minimum(cntf.astype(jnp.int32), s - 1)
        row = (lax.broadcasted_iota(jnp.int32, (r, 1), 0)
               + (pl.program_id(0) * _Q + q) * r)
        o_ref[q * r:(q + 1) * r, :] = row * s + cnt


def _median_indices(w2):
    B, S = w2.shape
    in_specs = [
        pl.BlockSpec((_R, S), functools.partial(lambda q, i: (i * _Q + q, 0), q))
        for q in range(_Q)
    ]
    return pl.pallas_call(
        _idx_body,
        grid=(B // (_R * _Q),),
        in_specs=in_specs,
        out_specs=pl.BlockSpec((_R * _Q, 1), lambda i: (i, 0)),
        out_shape=jax.ShapeDtypeStruct((B, 1), jnp.int32),
    )(*([w2] * _Q))


def _make_sc_gather(B):
    bpw = B // _NW
    nch = bpw // 128
    nrounds = nch // _CPR
    mesh = plsc.VectorSubcoreMesh(core_axis_name="c", subcore_axis_name="s")

    @functools.partial(
        pl.kernel,
        out_type=jax.ShapeDtypeStruct((_NW, nch, 128), jnp.float32),
        mesh=mesh,
        scratch_types=[
            pltpu.VMEM((nch, 128), jnp.int32),
            pltpu.VMEM((nch, 128), jnp.float32),
            pltpu.VMEM((nch, 128), jnp.float32),
            pltpu.SemaphoreType.DMA,
            pltpu.SemaphoreType.DMA,
        ],
    )
    def gather_kernel(gidx_hbm, s_hbm, e_hbm, out_hbm,
                      idx_v, sbuf, ebuf, sem_s, sem_e):
        wid = lax.axis_index("s") * 2 + lax.axis_index("c")
        pltpu.sync_copy(gidx_hbm.at[wid], idx_v)

        def gather_round(r, carry):
            handles = []
            for j in range(_CPR):
                c = r * _CPR + j
                handles.append(
                    pltpu.async_copy(s_hbm.at[idx_v.at[c]], sbuf.at[c], sem_s))
                handles.append(
                    pltpu.async_copy(e_hbm.at[idx_v.at[c]], ebuf.at[c], sem_e))
            for h in handles:
                h.wait()
            return carry

        lax.fori_loop(0, nrounds, gather_round, 0)

        def avg_row(i, carry):
            for j in range(8):
                sl = pl.ds(j * 16, 16)
                sbuf[i, sl] = (sbuf[i, sl] + ebuf[i, sl]) * 0.5
            return carry

        lax.fori_loop(0, nch, avg_row, 0)

        pltpu.sync_copy(sbuf, out_hbm.at[wid])

    return gather_kernel


def kernel(weights, starts, ends):
    B, S = weights.shape[0], weights.shape[1]
    w2 = weights.reshape(B, S)
    gidx = _median_indices(w2)
    gidx3 = gidx.reshape(_NW, -1, 128)
    sflat = starts.reshape(B * S)
    eflat = ends.reshape(B * S)
    out = _make_sc_gather(B)(gidx3, sflat, eflat)
    return out.reshape(B, 1)


# speedup vs baseline: 27.7743x; 1.2260x over previous
import functools

import jax
import jax.numpy as jnp
from jax import lax
from jax.experimental import pallas as pl
from jax.experimental.pallas import tpu as pltpu
from jax.experimental.pallas import tpu_sc as plsc

_R = 2048
_Q = 4

_NW = 32
_CPR = 8


def _idx_body(*refs):
    w_refs, o_ref = refs[:_Q], refs[_Q]
    s = w_refs[0].shape[1]
    r = w_refs[0].shape[0]
    tri = (
        lax.broadcasted_iota(jnp.int32, (s, s), 0)
        <= lax.broadcasted_iota(jnp.int32, (s, s), 1)
    ).astype(jnp.float32)
    for q in range(_Q):
        w = w_refs[q][...]
        cum = jnp.dot(
            w, tri, preferred_element_type=jnp.float32,
            precision=lax.Precision.HIGHEST,
        )
        cntf = jnp.sum(jnp.where(cum < 0.5, 1.0, 0.0), axis=1, keepdims=True)
        cntf = cntf.reshape(1, r)
        cnt = jnp.minimum(cntf.astype(jnp.int32), s - 1)
        row = (lax.broadcasted_iota(jnp.int32, (1, r), 1)
               + (pl.program_id(0) * _Q + q) * r)
        o_ref[0, q, :] = (row * s + cnt)[0, :]


def _median_indices(w2):
    B, S = w2.shape
    in_specs = [
        pl.BlockSpec((_R, S), functools.partial(lambda q, i: (i * _Q + q, 0), q))
        for q in range(_Q)
    ]
    nstep = B // (_R * _Q)
    return pl.pallas_call(
        _idx_body,
        grid=(nstep,),
        in_specs=in_specs,
        out_specs=pl.BlockSpec((1, _Q, _R), lambda i: (i, 0, 0)),
        out_shape=jax.ShapeDtypeStruct((nstep, _Q, _R), jnp.int32),
    )(*([w2] * _Q))


def _make_sc_gather(B):
    bpw = B // _NW
    nch = bpw // 128
    nrounds = nch // _CPR
    mesh = plsc.VectorSubcoreMesh(core_axis_name="c", subcore_axis_name="s")

    @functools.partial(
        pl.kernel,
        out_type=jax.ShapeDtypeStruct((_NW, nch, 128), jnp.float32),
        mesh=mesh,
        scratch_types=[
            pltpu.VMEM((nch, 128), jnp.int32),
            pltpu.VMEM((nch, 128), jnp.float32),
            pltpu.VMEM((nch, 128), jnp.float32),
            pltpu.SemaphoreType.DMA,
            pltpu.SemaphoreType.DMA,
        ],
    )
    def gather_kernel(gidx_hbm, s_hbm, e_hbm, out_hbm,
                      idx_v, sbuf, ebuf, sem_s, sem_e):
        wid = lax.axis_index("s") * 2 + lax.axis_index("c")
        pltpu.sync_copy(gidx_hbm.at[wid], idx_v)

        def gather_round(r, carry):
            handles = []
            for j in range(_CPR):
                c = r * _CPR + j
                handles.append(
                    pltpu.async_copy(s_hbm.at[idx_v.at[c]], sbuf.at[c], sem_s))
                handles.append(
                    pltpu.async_copy(e_hbm.at[idx_v.at[c]], ebuf.at[c], sem_e))
            for h in handles:
                h.wait()
            return carry

        lax.fori_loop(0, nrounds, gather_round, 0)

        def avg_row(i, carry):
            for j in range(8):
                sl = pl.ds(j * 16, 16)
                sbuf[i, sl] = (sbuf[i, sl] + ebuf[i, sl]) * 0.5
            return carry

        lax.fori_loop(0, nch, avg_row, 0)

        pltpu.sync_copy(sbuf, out_hbm.at[wid])

    return gather_kernel


def kernel(weights, starts, ends):
    B, S = weights.shape[0], weights.shape[1]
    w2 = weights.reshape(B, S)
    gidx = _median_indices(w2)
    gidx3 = gidx.reshape(_NW, -1, 128)
    sflat = starts.reshape(B * S)
    eflat = ends.reshape(B * S)
    out = _make_sc_gather(B)(gidx3, sflat, eflat)
    return out.reshape(B, 1)


# speedup vs baseline: 28.2683x; 1.0178x over previous
import functools

import jax
import jax.numpy as jnp
from jax import lax
from jax.experimental import pallas as pl
from jax.experimental.pallas import tpu as pltpu
from jax.experimental.pallas import tpu_sc as plsc

_R = 2048
_Q = 4

_NW = 32
_CPR = 8


def _idx_body(*refs):
    w_refs, o_ref = refs[:_Q], refs[_Q]
    s = w_refs[0].shape[1]
    r = w_refs[0].shape[0]
    tri = (
        lax.broadcasted_iota(jnp.int32, (s, s), 0)
        <= lax.broadcasted_iota(jnp.int32, (s, s), 1)
    ).astype(jnp.float32)
    for q in range(_Q):
        w = w_refs[q][...]
        cum = jnp.dot(
            w, tri, preferred_element_type=jnp.float32,
            precision=lax.Precision.HIGHEST,
        )
        cntf = jnp.sum(jnp.where(cum < 0.5, 1.0, 0.0), axis=1, keepdims=True)
        rq = r // s
        cntf = cntf.reshape(rq, s)
        cnt = jnp.minimum(cntf.astype(jnp.int32), s - 1)
        ray = (lax.broadcasted_iota(jnp.int32, (rq, s), 0) * s
               + lax.broadcasted_iota(jnp.int32, (rq, s), 1)
               + (pl.program_id(0) * _Q + q) * r)
        o_ref[0, q * rq:(q + 1) * rq, :] = ray * s + cnt


def _median_indices(w2):
    B, S = w2.shape
    in_specs = [
        pl.BlockSpec((_R, S), functools.partial(lambda q, i: (i * _Q + q, 0), q))
        for q in range(_Q)
    ]
    nstep = B // (_R * _Q)
    rows = _R * _Q // S
    return pl.pallas_call(
        _idx_body,
        grid=(nstep,),
        in_specs=in_specs,
        out_specs=pl.BlockSpec((1, rows, S), lambda i: (i, 0, 0)),
        out_shape=jax.ShapeDtypeStruct((nstep, rows, S), jnp.int32),
    )(*([w2] * _Q))


def _make_sc_gather(B):
    bpw = B // _NW
    nch = bpw // 128
    nrounds = nch // _CPR
    mesh = plsc.VectorSubcoreMesh(core_axis_name="c", subcore_axis_name="s")

    @functools.partial(
        pl.kernel,
        out_type=jax.ShapeDtypeStruct((_NW, nch, 128), jnp.float32),
        mesh=mesh,
        scratch_types=[
            pltpu.VMEM((nch, 128), jnp.int32),
            pltpu.VMEM((nch, 128), jnp.float32),
            pltpu.VMEM((nch, 128), jnp.float32),
            pltpu.SemaphoreType.DMA,
            pltpu.SemaphoreType.DMA,
        ],
    )
    def gather_kernel(gidx_hbm, s_hbm, e_hbm, out_hbm,
                      idx_v, sbuf, ebuf, sem_s, sem_e):
        wid = lax.axis_index("s") * 2 + lax.axis_index("c")
        pltpu.sync_copy(gidx_hbm.at[wid], idx_v)

        def gather_round(r, carry):
            handles = []
            for j in range(_CPR):
                c = r * _CPR + j
                handles.append(
                    pltpu.async_copy(s_hbm.at[idx_v.at[c]], sbuf.at[c], sem_s))
                handles.append(
                    pltpu.async_copy(e_hbm.at[idx_v.at[c]], ebuf.at[c], sem_e))
            for h in handles:
                h.wait()
            return carry

        lax.fori_loop(0, nrounds, gather_round, 0)

        def avg_row(i, carry):
            for j in range(8):
                sl = pl.ds(j * 16, 16)
                sbuf[i, sl] = (sbuf[i, sl] + ebuf[i, sl]) * 0.5
            return carry

        lax.fori_loop(0, nch, avg_row, 0)

        pltpu.sync_copy(sbuf, out_hbm.at[wid])

    return gather_kernel


def kernel(weights, starts, ends):
    B, S = weights.shape[0], weights.shape[1]
    w2 = weights.reshape(B, S)
    gidx = _median_indices(w2)
    gidx3 = gidx.reshape(_NW, -1, 128)
    sflat = starts.reshape(B * S)
    eflat = ends.reshape(B * S)
    out = _make_sc_gather(B)(gidx3, sflat, eflat)
    return out.reshape(B, 1)


# speedup vs baseline: 28.6988x; 1.0152x over previous
import functools

import jax
import jax.numpy as jnp
from jax import lax
from jax.experimental import pallas as pl
from jax.experimental.pallas import tpu as pltpu
from jax.experimental.pallas import tpu_sc as plsc

_R = 2048
_Q = 8

_NW = 32
_CPR = 8


def _idx_body(*refs):
    w_refs, o_ref = refs[:_Q], refs[_Q]
    s = w_refs[0].shape[1]
    r = w_refs[0].shape[0]
    tri = (
        lax.broadcasted_iota(jnp.int32, (s, s), 0)
        <= lax.broadcasted_iota(jnp.int32, (s, s), 1)
    ).astype(jnp.float32)
    for q in range(_Q):
        w = w_refs[q][...]
        cum = jnp.dot(
            w, tri, preferred_element_type=jnp.float32,
            precision=lax.Precision.HIGHEST,
        )
        cntf = jnp.sum(jnp.where(cum < 0.5, 1.0, 0.0), axis=1, keepdims=True)
        rq = r // s
        cntf = cntf.reshape(rq, s)
        cnt = jnp.minimum(cntf.astype(jnp.int32), s - 1)
        ray = (lax.broadcasted_iota(jnp.int32, (rq, s), 0) * s
               + lax.broadcasted_iota(jnp.int32, (rq, s), 1)
               + (pl.program_id(0) * _Q + q) * r)
        o_ref[0, q * rq:(q + 1) * rq, :] = ray * s + cnt


def _median_indices(w2):
    B, S = w2.shape
    in_specs = [
        pl.BlockSpec((_R, S), functools.partial(lambda q, i: (i * _Q + q, 0), q))
        for q in range(_Q)
    ]
    nstep = B // (_R * _Q)
    rows = _R * _Q // S
    return pl.pallas_call(
        _idx_body,
        grid=(nstep,),
        in_specs=in_specs,
        out_specs=pl.BlockSpec((1, rows, S), lambda i: (i, 0, 0)),
        out_shape=jax.ShapeDtypeStruct((nstep, rows, S), jnp.int32),
    )(*([w2] * _Q))


def _make_sc_gather(B):
    bpw = B // _NW
    nch = bpw // 128
    nrounds = nch // _CPR
    mesh = plsc.VectorSubcoreMesh(core_axis_name="c", subcore_axis_name="s")

    @functools.partial(
        pl.kernel,
        out_type=jax.ShapeDtypeStruct((_NW, nch, 128), jnp.float32),
        mesh=mesh,
        scratch_types=[
            pltpu.VMEM((nch, 128), jnp.int32),
            pltpu.VMEM((nch, 128), jnp.float32),
            pltpu.VMEM((nch, 128), jnp.float32),
            pltpu.SemaphoreType.DMA,
            pltpu.SemaphoreType.DMA,
        ],
    )
    def gather_kernel(gidx_hbm, s_hbm, e_hbm, out_hbm,
                      idx_v, sbuf, ebuf, sem_s, sem_e):
        wid = lax.axis_index("s") * 2 + lax.axis_index("c")
        pltpu.sync_copy(gidx_hbm.at[wid], idx_v)

        def gather_round(r, carry):
            handles = []
            for j in range(_CPR):
                c = r * _CPR + j
                handles.append(
                    pltpu.async_copy(s_hbm.at[idx_v.at[c]], sbuf.at[c], sem_s))
                handles.append(
                    pltpu.async_copy(e_hbm.at[idx_v.at[c]], ebuf.at[c], sem_e))
            for h in handles:
                h.wait()
            return carry

        lax.fori_loop(0, nrounds, gather_round, 0)

        def avg_row(i, carry):
            for j in range(8):
                sl = pl.ds(j * 16, 16)
                sbuf[i, sl] = (sbuf[i, sl] + ebuf[i, sl]) * 0.5
            return carry

        lax.fori_loop(0, nch, avg_row, 0)

        pltpu.sync_copy(sbuf, out_hbm.at[wid])

    return gather_kernel


def kernel(weights, starts, ends):
    B, S = weights.shape[0], weights.shape[1]
    w2 = weights.reshape(B, S)
    gidx = _median_indices(w2)
    gidx3 = gidx.reshape(_NW, -1, 128)
    sflat = starts.reshape(B * S)
    eflat = ends.reshape(B * S)
    out = _make_sc_gather(B)(gidx3, sflat, eflat)
    return out.reshape(B, 1)


# speedup vs baseline: 51.5025x; 1.7946x over previous
import functools

import jax
import jax.numpy as jnp
from jax import lax
from jax.experimental import pallas as pl
from jax.experimental.pallas import tpu as pltpu
from jax.experimental.pallas import tpu_sc as plsc

_R = 2048
_Q = 8

_NW = 32
_CPR = 8

_K = 16


def _idx_body(*refs):
    w_refs, o_ref = refs[:_Q], refs[_Q]
    s = w_refs[0].shape[1]
    r = w_refs[0].shape[0]
    tri = (
        lax.broadcasted_iota(jnp.int32, (s, s), 0)
        <= lax.broadcasted_iota(jnp.int32, (s, s), 1)
    ).astype(jnp.float32)
    for q in range(_Q):
        w = w_refs[q][...]
        cum = jnp.dot(
            w, tri, preferred_element_type=jnp.float32,
            precision=lax.Precision.HIGHEST,
        )
        cntf = jnp.sum(jnp.where(cum < 0.5, 1.0, 0.0), axis=1, keepdims=True)
        rq = r // s
        cntf = cntf.reshape(rq, s)
        cnt = jnp.minimum(cntf.astype(jnp.int32), s - 1)
        ray = (lax.broadcasted_iota(jnp.int32, (rq, s), 0) * s
               + lax.broadcasted_iota(jnp.int32, (rq, s), 1)
               + (pl.program_id(0) * _Q + q) * r)
        o_ref[0, q * rq:(q + 1) * rq, :] = ray * s + cnt


def _median_indices(w2):
    B, S = w2.shape
    in_specs = [
        pl.BlockSpec((_R, S), functools.partial(lambda q, i: (i * _Q + q, 0), q))
        for q in range(_Q)
    ]
    nstep = B // (_R * _Q)
    rows = _R * _Q // S
    return pl.pallas_call(
        _idx_body,
        grid=(nstep,),
        in_specs=in_specs,
        out_specs=pl.BlockSpec((1, rows, S), lambda i: (i, 0, 0)),
        out_shape=jax.ShapeDtypeStruct((nstep, rows, S), jnp.int32),
    )(*([w2] * _Q))


_NS = 4


def _make_sc_count(B, S):
    bpw = B // _NW
    nch = bpw // 128
    gpc = 128 // 16
    mesh = plsc.VectorSubcoreMesh(core_axis_name="c", subcore_axis_name="s")

    @functools.partial(
        pl.kernel,
        out_type=(jax.ShapeDtypeStruct((_NW, nch, 128), jnp.int32),
                  jax.ShapeDtypeStruct((_NW, 16), jnp.float32)),
        mesh=mesh,
        scratch_types=[
            pltpu.VMEM((_NS, 128, 128), jnp.float32),
            pltpu.VMEM((nch, 128), jnp.int32),
            pltpu.VMEM((16,), jnp.float32),
            pltpu.SemaphoreType.DMA((_NS,)),
        ],
        compiler_params=pltpu.CompilerParams(needs_layout_passes=False),
    )
    def count_kernel(w_hbm, gidx_hbm, min_hbm, wbuf, gbuf, minbuf, sems):
        wid = lax.axis_index("s") * 2 + lax.axis_index("c")
        base = wid * bpw
        lanes = lax.iota(jnp.int32, 16)

        def issue(c, slot):
            pltpu.async_copy(
                w_hbm.at[pl.ds(base + c * 128, 128)],
                wbuf.at[slot], sems.at[slot])

        def drain(c, slot):
            pltpu.make_async_copy(
                w_hbm.at[pl.ds(base + c * 128, 128)],
                wbuf.at[slot], sems.at[slot]).wait()

        for c in range(_NS):
            issue(c, c)

        def chunk_body(c, min_v):
            slot = c % _NS
            drain(c, slot)

            for g in range(gpc):
                rsv = (base + c * 128 + g * 16 + lanes) * S
                cacc = jnp.zeros((16,), jnp.int32)
                for r in range(16):
                    v = wbuf[slot, g * 16 + r, pl.ds(0, _K)]
                    cum = jnp.cumsum(v)
                    cntv = plsc.all_reduce_population_count(cum < 0.5)
                    cacc = jnp.where(lanes == r, cntv, cacc)
                    min_v = jnp.minimum(min_v, cum)
                gbuf[c, pl.ds(g * 16, 16)] = rsv + cacc

            @pl.when(c + _NS < nch)
            def _():
                issue(c + _NS, slot)

            return min_v

        min_v = lax.fori_loop(
            0, nch, chunk_body,
            jnp.full((16,), jnp.finfo(jnp.float32).max, jnp.float32))
        minbuf[...] = min_v
        pltpu.sync_copy(gbuf, gidx_hbm.at[wid])
        pltpu.sync_copy(minbuf, min_hbm.at[wid])

    return count_kernel


def _make_sc_gather(B):
    bpw = B // _NW
    nch = bpw // 128
    nrounds = nch // _CPR
    mesh = plsc.VectorSubcoreMesh(core_axis_name="c", subcore_axis_name="s")

    @functools.partial(
        pl.kernel,
        out_type=jax.ShapeDtypeStruct((_NW, nch, 128), jnp.float32),
        mesh=mesh,
        scratch_types=[
            pltpu.VMEM((nch, 128), jnp.int32),
            pltpu.VMEM((nch, 128), jnp.float32),
            pltpu.VMEM((nch, 128), jnp.float32),
            pltpu.SemaphoreType.DMA,
            pltpu.SemaphoreType.DMA,
        ],
    )
    def gather_kernel(gidx_hbm, s_hbm, e_hbm, out_hbm,
                      idx_v, sbuf, ebuf, sem_s, sem_e):
        wid = lax.axis_index("s") * 2 + lax.axis_index("c")
        pltpu.sync_copy(gidx_hbm.at[wid], idx_v)

        def gather_round(r, carry):
            handles = []
            for j in range(_CPR):
                c = r * _CPR + j
                handles.append(
                    pltpu.async_copy(s_hbm.at[idx_v.at[c]], sbuf.at[c], sem_s))
                handles.append(
                    pltpu.async_copy(e_hbm.at[idx_v.at[c]], ebuf.at[c], sem_e))
            for h in handles:
                h.wait()
            return carry

        lax.fori_loop(0, nrounds, gather_round, 0)

        def avg_row(i, carry):
            for j in range(8):
                sl = pl.ds(j * 16, 16)
                sbuf[i, sl] = (sbuf[i, sl] + ebuf[i, sl]) * 0.5
            return carry

        lax.fori_loop(0, nch, avg_row, 0)

        pltpu.sync_copy(sbuf, out_hbm.at[wid])

    return gather_kernel


def kernel(weights, starts, ends):
    B, S = weights.shape[0], weights.shape[1]
    w2 = weights.reshape(B, S)
    gfast, mins = _make_sc_count(B, S)(w2)
    need_full = jnp.min(mins[:, 15]) < 0.5
    gidx3 = lax.cond(
        need_full,
        lambda: _median_indices(w2).reshape(_NW, B // (_NW * 128), 128),
        lambda: gfast,
    )
    sflat = starts.reshape(B * S)
    eflat = ends.reshape(B * S)
    out = _make_sc_gather(B)(gidx3, sflat, eflat)
    return out.reshape(B, 1)


# speedup vs baseline: 62.7459x; 1.2183x over previous
import functools

import jax
import jax.numpy as jnp
from jax import lax
from jax.experimental import pallas as pl
from jax.experimental.pallas import tpu as pltpu
from jax.experimental.pallas import tpu_sc as plsc

_R = 2048
_Q = 8

_NW = 32
_CPR = 8

_K = 16


def _idx_body(*refs):
    w_refs, o_ref = refs[:_Q], refs[_Q]
    s = w_refs[0].shape[1]
    r = w_refs[0].shape[0]
    tri = (
        lax.broadcasted_iota(jnp.int32, (s, s), 0)
        <= lax.broadcasted_iota(jnp.int32, (s, s), 1)
    ).astype(jnp.float32)
    for q in range(_Q):
        w = w_refs[q][...]
        cum = jnp.dot(
            w, tri, preferred_element_type=jnp.float32,
            precision=lax.Precision.HIGHEST,
        )
        cntf = jnp.sum(jnp.where(cum < 0.5, 1.0, 0.0), axis=1, keepdims=True)
        rq = r // s
        cntf = cntf.reshape(rq, s)
        cnt = jnp.minimum(cntf.astype(jnp.int32), s - 1)
        ray = (lax.broadcasted_iota(jnp.int32, (rq, s), 0) * s
               + lax.broadcasted_iota(jnp.int32, (rq, s), 1)
               + (pl.program_id(0) * _Q + q) * r)
        o_ref[0, q * rq:(q + 1) * rq, :] = ray * s + cnt


def _median_indices(w2):
    B, S = w2.shape
    in_specs = [
        pl.BlockSpec((_R, S), functools.partial(lambda q, i: (i * _Q + q, 0), q))
        for q in range(_Q)
    ]
    nstep = B // (_R * _Q)
    rows = _R * _Q // S
    return pl.pallas_call(
        _idx_body,
        grid=(nstep,),
        in_specs=in_specs,
        out_specs=pl.BlockSpec((1, rows, S), lambda i: (i, 0, 0)),
        out_shape=jax.ShapeDtypeStruct((nstep, rows, S), jnp.int32),
    )(*([w2] * _Q))


_NS = 4


_GD = 8


def _make_sc_fused(B, S):
    bpw = B // _NW
    nch = bpw // 128
    gpc = 128 // 16
    mesh = plsc.VectorSubcoreMesh(core_axis_name="c", subcore_axis_name="s")

    @functools.partial(
        pl.kernel,
        out_type=(jax.ShapeDtypeStruct((_NW, nch, 128), jnp.float32),
                  jax.ShapeDtypeStruct((_NW, 16), jnp.float32)),
        mesh=mesh,
        scratch_types=[
            pltpu.VMEM((_NS, 128, 128), jnp.float32),
            pltpu.VMEM((nch, 128), jnp.int32),
            pltpu.VMEM((nch, 128), jnp.float32),
            pltpu.VMEM((nch, 128), jnp.float32),
            pltpu.VMEM((16,), jnp.float32),
            pltpu.SemaphoreType.DMA((_NS,)),
            pltpu.SemaphoreType.DMA((_GD,)),
            pltpu.SemaphoreType.DMA((_GD,)),
        ],
        compiler_params=pltpu.CompilerParams(needs_layout_passes=False),
    )
    def fused_kernel(w_hbm, s_hbm, e_hbm, out_hbm, min_hbm,
                     wbuf, idxb, sbuf, ebuf, minbuf, wsems, gsem_s, gsem_e):
        wid = lax.axis_index("s") * 2 + lax.axis_index("c")
        base = wid * bpw
        lanes = lax.iota(jnp.int32, 16)

        def issue_w(c, slot):
            pltpu.async_copy(
                w_hbm.at[pl.ds(base + c * 128, 128)],
                wbuf.at[slot], wsems.at[slot])

        def drain_w(c, slot):
            pltpu.make_async_copy(
                w_hbm.at[pl.ds(base + c * 128, 128)],
                wbuf.at[slot], wsems.at[slot]).wait()

        def issue_g(c):
            g = c % _GD
            pltpu.async_copy(s_hbm.at[idxb.at[c]], sbuf.at[c], gsem_s.at[g])
            pltpu.async_copy(e_hbm.at[idxb.at[c]], ebuf.at[c], gsem_e.at[g])

        def finish_g(c):
            g = c % _GD
            pltpu.make_async_copy(
                s_hbm.at[idxb.at[c]], sbuf.at[c], gsem_s.at[g]).wait()
            pltpu.make_async_copy(
                e_hbm.at[idxb.at[c]], ebuf.at[c], gsem_e.at[g]).wait()
            for j in range(8):
                sl = pl.ds(j * 16, 16)
                sbuf[c, sl] = (sbuf[c, sl] + ebuf[c, sl]) * 0.5

        for c in range(_NS):
            issue_w(c, c)

        def chunk_body(c, min_v):
            slot = c % _NS
            drain_w(c, slot)

            for g in range(gpc):
                rsv = (base + c * 128 + g * 16 + lanes) * S
                cacc = jnp.zeros((16,), jnp.int32)
                for r in range(16):
                    v = wbuf[slot, g * 16 + r, pl.ds(0, _K)]
                    cum = jnp.cumsum(v)
                    cntv = plsc.all_reduce_population_count(cum < 0.5)
                    cacc = jnp.where(lanes == r, cntv, cacc)
                    min_v = jnp.minimum(min_v, cum)
                idxb[c, pl.ds(g * 16, 16)] = rsv + cacc

            @pl.when(c >= _GD)
            def _():
                finish_g(c - _GD)

            issue_g(c)

            @pl.when(c + _NS < nch)
            def _():
                issue_w(c + _NS, slot)

            return min_v

        min_v = lax.fori_loop(
            0, nch, chunk_body,
            jnp.full((16,), jnp.finfo(jnp.float32).max, jnp.float32))

        def tail_body(c, carry):
            finish_g(c)
            return carry

        lax.fori_loop(nch - _GD, nch, tail_body, 0)

        minbuf[...] = min_v
        pltpu.sync_copy(sbuf, out_hbm.at[wid])
        pltpu.sync_copy(minbuf, min_hbm.at[wid])

    return fused_kernel


def _make_sc_gather(B):
    bpw = B // _NW
    nch = bpw // 128
    nrounds = nch // _CPR
    mesh = plsc.VectorSubcoreMesh(core_axis_name="c", subcore_axis_name="s")

    @functools.partial(
        pl.kernel,
        out_type=jax.ShapeDtypeStruct((_NW, nch, 128), jnp.float32),
        mesh=mesh,
        scratch_types=[
            pltpu.VMEM((nch, 128), jnp.int32),
            pltpu.VMEM((nch, 128), jnp.float32),
            pltpu.VMEM((nch, 128), jnp.float32),
            pltpu.SemaphoreType.DMA,
            pltpu.SemaphoreType.DMA,
        ],
        compiler_params=pltpu.CompilerParams(needs_layout_passes=False),
    )
    def gather_kernel(gidx_hbm, s_hbm, e_hbm, out_hbm,
                      idx_v, sbuf, ebuf, sem_s, sem_e):
        wid = lax.axis_index("s") * 2 + lax.axis_index("c")
        pltpu.sync_copy(gidx_hbm.at[wid], idx_v)

        def gather_round(r, carry):
            handles = []
            for j in range(_CPR):
                c = r * _CPR + j
                handles.append(
                    pltpu.async_copy(s_hbm.at[idx_v.at[c]], sbuf.at[c], sem_s))
                handles.append(
                    pltpu.async_copy(e_hbm.at[idx_v.at[c]], ebuf.at[c], sem_e))
            for h in handles:
                h.wait()
            return carry

        lax.fori_loop(0, nrounds, gather_round, 0)

        def avg_row(i, carry):
            for j in range(8):
                sl = pl.ds(j * 16, 16)
                sbuf[i, sl] = (sbuf[i, sl] + ebuf[i, sl]) * 0.5
            return carry

        lax.fori_loop(0, nch, avg_row, 0)

        pltpu.sync_copy(sbuf, out_hbm.at[wid])

    return gather_kernel


def kernel(weights, starts, ends):
    B, S = weights.shape[0], weights.shape[1]
    w2 = weights.reshape(B, S)
    sflat = starts.reshape(B * S)
    eflat = ends.reshape(B * S)
    out_fast, mins = _make_sc_fused(B, S)(w2, sflat, eflat)
    need_full = jnp.min(mins[:, 15]) < 0.5
    out = lax.cond(
        need_full,
        lambda: _make_sc_gather(B)(
            _median_indices(w2).reshape(_NW, B // (_NW * 128), 128),
            sflat, eflat),
        lambda: out_fast,
    )
    return out.reshape(B, 1)
